# Initial kernel scaffold; baseline (speedup 1.0000x reference)
import jax
import jax.numpy as jnp
from jax.experimental import pallas as pl


def kernel(x, weight):
    raise NotImplementedError("write your pallas kernel here")


# speedup vs baseline: 1.3567x; 1.3567x over previous
import functools
import math

import jax
import jax.numpy as jnp
from jax import lax
from jax.experimental import pallas as pl
from jax.experimental.pallas import tpu as pltpu
from jax.experimental.pallas import tpu_sc as plsc

EMB = 32
NCORES = 2
NSUB = 16
NW = NCORES * NSUB
SCALE = math.sqrt(EMB)

CHUNK = 512
SUB = 128
K = CHUNK // SUB


@functools.lru_cache(maxsize=None)
def _build(B, V):
    bpw = B // NW
    iters = bpw // CHUNK
    mesh = plsc.VectorSubcoreMesh(
        core_axis_name="c", subcore_axis_name="s",
        num_cores=NCORES, num_subcores=NSUB)

    def body(w_hbm, idx_hbm, out_hbm, idx_v, rows_v, sem):
        wid = lax.axis_index("s") * NCORES + lax.axis_index("c")
        sub_base = wid * (bpw // SUB)
        row_base = wid * bpw

        def chunk_step(g, carry):
            pltpu.sync_copy(idx_hbm.at[pl.ds(sub_base + g * K, K)], idx_v)
            descs = [
                pltpu.async_copy(
                    w_hbm.at[idx_v.at[j]],
                    rows_v.at[pl.ds(j * SUB, SUB)], sem)
                for j in range(K)
            ]
            for d in descs:
                d.wait()

            def scale_step(i, c):
                base = i * 8
                for r in range(8):
                    for h in range(2):
                        sl = (base + r, pl.ds(h * 16, 16))
                        rows_v[sl] = rows_v[sl] * SCALE
                return c

            lax.fori_loop(0, CHUNK // 8, scale_step, 0)

            pltpu.sync_copy(rows_v, out_hbm.at[pl.ds(row_base + g * CHUNK,
                                                     CHUNK)])
            return carry

        lax.fori_loop(0, iters, chunk_step, 0)

    return pl.kernel(
        body,
        out_type=jax.ShapeDtypeStruct((B, EMB), jnp.float32),
        mesh=mesh,
        compiler_params=pltpu.CompilerParams(use_tc_tiling_on_sc=False),
        scratch_types=[
            pltpu.VMEM((K, SUB), jnp.int32),
            pltpu.VMEM((CHUNK, EMB), jnp.float32),
            pltpu.SemaphoreType.DMA,
        ],
    )


def kernel(x, weight):
    B0, H = x.shape
    B = B0 * H
    idx2d = x.reshape(B // SUB, SUB).astype(jnp.int32)
    out = _build(B, weight.shape[0])(weight, idx2d)
    return out.reshape(B0, H, EMB)


# speedup vs baseline: 1.4676x; 1.0818x over previous
import functools
import math

import jax
import jax.numpy as jnp
from jax import lax
from jax.experimental import pallas as pl
from jax.experimental.pallas import tpu as pltpu
from jax.experimental.pallas import tpu_sc as plsc

EMB = 32
NCORES = 2
NSUB = 16
NW = NCORES * NSUB
SCALE = math.sqrt(EMB)

CHUNK = 512
SUB = 128
K = CHUNK // SUB
NBUF = 4
LEAD = 2


@functools.lru_cache(maxsize=None)
def _build(B, V):
    bpw = B // NW
    iters = bpw // CHUNK
    assert iters > NBUF and (iters - LEAD) % NBUF == 0
    mesh = plsc.VectorSubcoreMesh(
        core_axis_name="c", subcore_axis_name="s",
        num_cores=NCORES, num_subcores=NSUB)

    def body(w_hbm, idx_hbm, out_hbm, idx_v, rows_v,
             g0, g1, g2, g3, o0, o1, o2, o3):
        gsem = [g0, g1, g2, g3]
        osem = [o0, o1, o2, o3]
        wid = lax.axis_index("s") * NCORES + lax.axis_index("c")
        sub_base = wid * (bpw // SUB)
        row_base = wid * bpw

        def gather_descs(c, b):
            return [
                pltpu.make_async_copy(
                    w_hbm.at[idx_v.at[b].at[j]],
                    rows_v.at[b].at[pl.ds(j * SUB, SUB)],
                    gsem[b])
                for j in range(K)
            ]

        def fire(c, b):
            pltpu.sync_copy(idx_hbm.at[pl.ds(sub_base + c * K, K)],
                            idx_v.at[b])
            for d in gather_descs(c, b):
                d.start()

        def drain_gather(c, b):
            for d in gather_descs(c, b):
                d.wait()

        def out_desc(c, b):
            return pltpu.make_async_copy(
                rows_v.at[b],
                out_hbm.at[pl.ds(row_base + c * CHUNK, CHUNK)],
                osem[b])

        def scale(b):
            def scale_step(i, carry):
                base = i * 8
                for r in range(8):
                    for h in range(2):
                        sl = pl.ds(h * 16, 16)
                        rows_v[b, base + r, sl] = rows_v[b, base + r, sl] * SCALE
                return carry

            lax.fori_loop(0, CHUNK // 8, scale_step, 0)

        def process(c, b):
            drain_gather(c, b)
            scale(b)
            out_desc(c, b).start()

        for b in range(LEAD):
            fire(b, b)

        def step(g, carry):
            for i in range(NBUF):
                c = g * NBUF + i
                b = i
                process(c, b)
                c2 = c + LEAD
                b2 = (i + LEAD) % NBUF
                @pl.when(c2 >= NBUF)
                def _():
                    out_desc(c2 - NBUF, b2).wait()

                @pl.when(c2 < iters)
                def _():
                    fire(c2, b2)
            return carry

        lax.fori_loop(0, (iters - LEAD) // NBUF, step, 0)

        for i in range(LEAD):
            c = iters - LEAD + i
            process(c, c % NBUF)
        for i in range(NBUF):
            c = iters - NBUF + i
            out_desc(c, c % NBUF).wait()

    return pl.kernel(
        body,
        out_type=jax.ShapeDtypeStruct((B, EMB), jnp.float32),
        mesh=mesh,
        compiler_params=pltpu.CompilerParams(use_tc_tiling_on_sc=False),
        scratch_types=[
            pltpu.VMEM((NBUF, K, SUB), jnp.int32),
            pltpu.VMEM((NBUF, CHUNK, EMB), jnp.float32),
        ] + [pltpu.SemaphoreType.DMA] * (2 * NBUF),
    )


def kernel(x, weight):
    B0, H = x.shape
    B = B0 * H
    idx2d = x.reshape(B // SUB, SUB).astype(jnp.int32)
    out = _build(B, weight.shape[0])(weight, idx2d)
    return out.reshape(B0, H, EMB)


# speedup vs baseline: 1.4734x; 1.0039x over previous
import functools
import math

import jax
import jax.numpy as jnp
from jax import lax
from jax.experimental import pallas as pl
from jax.experimental.pallas import tpu as pltpu
from jax.experimental.pallas import tpu_sc as plsc

EMB = 32
NCORES = 2
NSUB = 16
NW = NCORES * NSUB
SCALE = math.sqrt(EMB)

SUB = 128
CHUNK = 640
K = CHUNK // SUB
NBUF = 5
LEAD = 2


@functools.lru_cache(maxsize=None)
def _build(B0, H, V):
    rows_w = B0 * H // NW
    bats_w = rows_w // H
    iters = rows_w // CHUNK
    outer = iters // NBUF
    bats_g = CHUNK * NBUF // H
    assert rows_w % CHUNK == 0 and iters % NBUF == 0 and LEAD < NBUF
    assert (CHUNK * NBUF) % H == 0
    SPLITS = []
    for s in range(NBUF):
        start = s * CHUNK
        pieces = []
        o = 0
        h = start % H
        db = start // H
        while o < CHUNK:
            n = min(H - h, CHUNK - o)
            pieces.append((o, n, db, h))
            o += n
            h = 0
            db += 1
        SPLITS.append(pieces)

    mesh = plsc.VectorSubcoreMesh(
        core_axis_name="c", subcore_axis_name="s",
        num_cores=NCORES, num_subcores=NSUB)

    def body(w_hbm, idx_hbm, out_hbm, idx_v, rows_v,
             g0, g1, g2, g3, g4, o0, o1, o2, o3, o4):
        gsem = [g0, g1, g2, g3, g4]
        osem = [o0, o1, o2, o3, o4]
        wid = lax.axis_index("s") * NCORES + lax.axis_index("c")
        sub_base = wid * (rows_w // SUB)
        bat_base = wid * bats_w

        def gather_descs(b):
            return [
                pltpu.make_async_copy(
                    w_hbm.at[idx_v.at[b, j]],
                    rows_v.at[b, pl.ds(j * SUB, SUB)],
                    gsem[b])
                for j in range(K)
            ]

        def fire(c, b):
            pltpu.sync_copy(idx_hbm.at[pl.ds(sub_base + c * K, K)],
                            idx_v.at[b])
            for d in gather_descs(b):
                d.start()

        def out_descs(g, s):
            b0 = bat_base + g * bats_g
            return [
                pltpu.make_async_copy(
                    rows_v.at[s, pl.ds(o, n)],
                    out_hbm.at[b0 + db, pl.ds(h0, n)],
                    osem[s])
                for (o, n, db, h0) in SPLITS[s]
            ]

        def scale(b):
            def scale_step(i, carry):
                base = i * 8
                for r in range(8):
                    for h in range(2):
                        sl = pl.ds(h * 16, 16)
                        rows_v[b, base + r, sl] = rows_v[b, base + r, sl] * SCALE
                return carry

            lax.fori_loop(0, CHUNK // 8, scale_step, 0)

        for b in range(LEAD):
            fire(b, b)

        def step(g, carry):
            for s in range(NBUF):
                c = g * NBUF + s
                b = s
                for d in gather_descs(b):
                    d.wait()
                scale(b)
                for d in out_descs(g, s):
                    d.start()
                c2 = c + LEAD
                s2 = (s + LEAD) % NBUF
                sp = (s + LEAD - NBUF) % NBUF
                gp = g + (s + LEAD - NBUF) // NBUF

                @pl.when(c2 >= NBUF)
                def _():
                    for d in out_descs(gp, sp):
                        d.wait()

                @pl.when(c2 < iters)
                def _():
                    fire(c2, s2)
            return carry

        lax.fori_loop(0, outer, step, 0)

        for s in range(LEAD, NBUF):
            for d in out_descs(outer - 1, s):
                d.wait()

    return pl.kernel(
        body,
        out_type=jax.ShapeDtypeStruct((B0, H, EMB), jnp.float32),
        mesh=mesh,
        compiler_params=pltpu.CompilerParams(use_tc_tiling_on_sc=False),
        scratch_types=[
            pltpu.VMEM((NBUF, K, SUB), jnp.int32),
            pltpu.VMEM((NBUF, CHUNK, EMB), jnp.float32),
        ] + [pltpu.SemaphoreType.DMA] * (2 * NBUF),
    )


def kernel(x, weight):
    B0, H = x.shape
    idx2d = x.reshape(B0 * H // SUB, SUB).astype(jnp.int32)
    return _build(B0, H, weight.shape[0])(weight, idx2d)
